# Initial kernel scaffold; baseline (speedup 1.0000x reference)
import jax
import jax.numpy as jnp
from jax.experimental import pallas as pl


def kernel(updates, mask):
    raise NotImplementedError("write your pallas kernel here")


# speedup vs baseline: 26.6390x; 26.6390x over previous
import functools

import jax
import jax.numpy as jnp
from jax import lax
from jax.experimental import pallas as pl
from jax.experimental.pallas import tpu as pltpu
from jax.experimental.pallas import tpu_sc as plsc

H = 256
W = 256
C = 96
NIN = H * W
NOUT = 4 * NIN
SLOTS = 6
PASSES = 8
NTEC = 16
ELEMS_PER_TEC = 24576
CHUNK = 12288
ACC_WORDS = SLOTS * NOUT
ACC_SLICE = ACC_WORDS // NTEC
ZCHUNK = 4096


def _pre_body(upd_ref, mask_ref, valT_ref, flatT_ref):
    u = upd_ref[...]
    m = mask_ref[...]
    c = lax.broadcasted_iota(jnp.int32, m.shape, 1)
    flat = m // C + (c % SLOTS) * NOUT
    valT_ref[...] = u.T
    flatT_ref[...] = flat.T


_PRE_BLK = 2048


def _pre(upd2d, mask2d):
    grid = (NIN // _PRE_BLK,)
    return pl.pallas_call(
        _pre_body,
        grid=grid,
        in_specs=[
            pl.BlockSpec((_PRE_BLK, C), lambda i: (i, 0)),
            pl.BlockSpec((_PRE_BLK, C), lambda i: (i, 0)),
        ],
        out_specs=[
            pl.BlockSpec((C, _PRE_BLK), lambda i: (0, i)),
            pl.BlockSpec((C, _PRE_BLK), lambda i: (0, i)),
        ],
        out_shape=[
            jax.ShapeDtypeStruct((C, NIN), jnp.float32),
            jax.ShapeDtypeStruct((C, NIN), jnp.int32),
        ],
    )(upd2d, mask2d)


def _sc_body(flatT_hbm, valT_hbm, zeros_hbm, out_hbm, idx_v, val_v, zer_v, acc_sh):
    s = lax.axis_index("c")
    t = lax.axis_index("s")
    pltpu.sync_copy(zeros_hbm, zer_v)
    for q in range(PASSES):
        for i in range(ACC_SLICE // ZCHUNK):
            pltpu.sync_copy(zer_v, acc_sh.at[pl.ds(t * ACC_SLICE + i * ZCHUNK, ZCHUNK)])
        plsc.subcore_barrier()
        elem0 = (12 * q + 6 * s) * NIN + t * ELEMS_PER_TEC
        for j in range(ELEMS_PER_TEC // CHUNK):
            pltpu.sync_copy(flatT_hbm.at[pl.ds(elem0 + j * CHUNK, CHUNK)], idx_v)
            pltpu.sync_copy(valT_hbm.at[pl.ds(elem0 + j * CHUNK, CHUNK)], val_v)
            pltpu.sync_copy(val_v, acc_sh.at[idx_v], add=True)
        plsc.subcore_barrier()
        out_base = (12 * q + 6 * s) * NOUT + t * ACC_SLICE
        pltpu.sync_copy(
            acc_sh.at[pl.ds(t * ACC_SLICE, ACC_SLICE)],
            out_hbm.at[pl.ds(out_base, ACC_SLICE)],
        )


def _sc_scatter(flatT, valT, zeros):
    mesh = plsc.VectorSubcoreMesh(core_axis_name="c", subcore_axis_name="s")
    fn = functools.partial(
        pl.kernel,
        mesh=mesh,
        out_type=jax.ShapeDtypeStruct((C * NOUT,), jnp.float32),
        scratch_types=[
            pltpu.VMEM((CHUNK,), jnp.int32),
            pltpu.VMEM((CHUNK,), jnp.float32),
            pltpu.VMEM((ZCHUNK,), jnp.float32),
            pltpu.VMEM_SHARED((ACC_WORDS,), jnp.float32),
        ],
    )(_sc_body)
    return fn(flatT, valT, zeros)


def _post_body(outT_ref, out_ref):
    out_ref[...] = outT_ref[...].T


_POST_BLK = 2048


def _post(outT):
    grid = (NOUT // _POST_BLK,)
    return pl.pallas_call(
        _post_body,
        grid=grid,
        in_specs=[pl.BlockSpec((C, _POST_BLK), lambda i: (0, i))],
        out_specs=pl.BlockSpec((_POST_BLK, C), lambda i: (i, 0)),
        out_shape=jax.ShapeDtypeStruct((NOUT, C), jnp.float32),
    )(outT)


def kernel(updates, mask):
    upd2d = updates.reshape(NIN, C)
    mask2d = mask.astype(jnp.int32).reshape(NIN, C)
    valT, flatT = _pre(upd2d, mask2d)
    zeros = jnp.zeros((ZCHUNK,), jnp.float32)
    outT = _sc_scatter(flatT.reshape(C * NIN), valT.reshape(C * NIN), zeros)
    out2d = _post(outT.reshape(C, NOUT))
    return out2d.reshape(1, 2 * H, 2 * W, C)


# speedup vs baseline: 46.0106x; 1.7272x over previous
import functools

import jax
import jax.numpy as jnp
from jax import lax
from jax.experimental import pallas as pl
from jax.experimental.pallas import tpu as pltpu
from jax.experimental.pallas import tpu_sc as plsc

H = 256
W = 256
C = 96
NIN = H * W
NOUT = 4 * NIN
NEL = NIN * C
SLOTS = 6
PASSES = 8
NTEC = 16
RUN = SLOTS * W
RUNS_PER_CHUNK = 8
CHUNK = RUN * RUNS_PER_CHUNK
ACC_WORDS = SLOTS * NOUT
ACC_SLICE = ACC_WORDS // NTEC
ZCHUNK = 4096
WO = 2 * W
HO = 2 * H
OROW = C * WO
DUMP_RUN = SLOTS * WO
DUMP_H = HO // NTEC


def _pre_body(mask_ref, flat_ref):
    i = pl.program_id(0)
    m = mask_ref[...]
    r = lax.broadcasted_iota(jnp.int32, m.shape, 0) + i * _PRE_ROWS
    slot = (r // 2) % SLOTS
    p = m // C
    flat_ref[...] = (p >> 9) * (SLOTS * WO) + slot * WO + (p & (WO - 1))


_PRE_ROWS = 1536


def _pre(mask2d):
    grid = (NEL // 128 // _PRE_ROWS,)
    return pl.pallas_call(
        _pre_body,
        grid=grid,
        in_specs=[pl.BlockSpec((_PRE_ROWS, 128), lambda i: (i, 0))],
        out_specs=pl.BlockSpec((_PRE_ROWS, 128), lambda i: (i, 0)),
        out_shape=jax.ShapeDtypeStruct((NEL // 128, 128), jnp.int32),
    )(mask2d)


def _sc_body(flat_hbm, upd_hbm, zeros_hbm, out_hbm, idx_v, val_v, zer_v,
             acc_sh, sem_in, sem_out):
    s = lax.axis_index("c")
    t = lax.axis_index("s")
    pltpu.sync_copy(zeros_hbm, zer_v)
    for q in range(PASSES):
        c0 = SLOTS * (2 * q + s)
        for i in range(ACC_SLICE // ZCHUNK):
            pltpu.sync_copy(zer_v, acc_sh.at[pl.ds(t * ACC_SLICE + i * ZCHUNK, ZCHUNK)])
        plsc.subcore_barrier()
        for j in range(2):
            h0 = NTEC * t + RUNS_PER_CHUNK * j
            waits = []
            for k in range(RUNS_PER_CHUNK):
                src = (h0 + k) * (C * W) + c0 * W
                waits.append(pltpu.async_copy(
                    flat_hbm.at[pl.ds(src, RUN)],
                    idx_v.at[pl.ds(k * RUN, RUN)], sem_in))
                waits.append(pltpu.async_copy(
                    upd_hbm.at[pl.ds(src, RUN)],
                    val_v.at[pl.ds(k * RUN, RUN)], sem_in))
            for wtt in waits:
                wtt.wait()
            pltpu.sync_copy(val_v, acc_sh.at[idx_v], add=True)
        plsc.subcore_barrier()
        waits = []
        for k in range(DUMP_H):
            h2 = DUMP_H * t + k
            waits.append(pltpu.async_copy(
                acc_sh.at[pl.ds(h2 * DUMP_RUN, DUMP_RUN)],
                out_hbm.at[pl.ds(h2 * OROW + c0 * WO, DUMP_RUN)], sem_out))
        for wtt in waits:
            wtt.wait()


def _sc_scatter(flat, upd, zeros):
    mesh = plsc.VectorSubcoreMesh(core_axis_name="c", subcore_axis_name="s")
    fn = functools.partial(
        pl.kernel,
        mesh=mesh,
        out_type=jax.ShapeDtypeStruct((HO * C * WO,), jnp.float32),
        scratch_types=[
            pltpu.VMEM((CHUNK,), jnp.int32),
            pltpu.VMEM((CHUNK,), jnp.float32),
            pltpu.VMEM((ZCHUNK,), jnp.float32),
            pltpu.VMEM_SHARED((ACC_WORDS,), jnp.float32),
            pltpu.SemaphoreType.DMA,
            pltpu.SemaphoreType.DMA,
        ],
    )(_sc_body)
    return fn(flat, upd, zeros)


def kernel(updates, mask):
    upd_n = updates.reshape(H, W, C).transpose(0, 2, 1).reshape(NEL)
    mask_n = (
        mask.astype(jnp.int32).reshape(H, W, C).transpose(0, 2, 1)
        .reshape(NEL // 128, 128)
    )
    flat = _pre(mask_n)
    zeros = jnp.zeros((ZCHUNK,), jnp.float32)
    out = _sc_scatter(flat.reshape(NEL), upd_n, zeros)
    return (
        out.reshape(HO, C, WO).transpose(0, 2, 1).reshape(1, HO, WO, C)
    )


# speedup vs baseline: 49.8410x; 1.0833x over previous
import functools

import jax
import jax.numpy as jnp
from jax import lax
from jax.experimental import pallas as pl
from jax.experimental.pallas import tpu as pltpu
from jax.experimental.pallas import tpu_sc as plsc

H = 256
W = 256
C = 96
NIN = H * W
NOUT = 4 * NIN
NEL = NIN * C
SLOTS = 6
PASSES = 8
NTEC = 16
RUN = SLOTS * W
RUNS_PER_CHUNK = 4
CHUNK = RUN * RUNS_PER_CHUNK
NCHUNK = 4
ACC_WORDS = SLOTS * NOUT
ACC_SLICE = ACC_WORDS // NTEC
ZCHUNK = 4096
WO = 2 * W
HO = 2 * H
OROW = C * WO
DUMP_RUN = SLOTS * WO
DUMP_H = HO // NTEC


def _pre_body(mask_ref, flat_ref):
    i = pl.program_id(0)
    m = mask_ref[...]
    r = lax.broadcasted_iota(jnp.int32, m.shape, 0) + i * _PRE_ROWS
    slot = (r // 2) % SLOTS
    p = ((m >> 5).astype(jnp.float32) * jnp.float32(0.33333334)).astype(jnp.int32)
    flat_ref[...] = (p >> 9) * (SLOTS * WO) + slot * WO + (p & (WO - 1))


_PRE_ROWS = 1536


def _pre(mask2d):
    grid = (NEL // 128 // _PRE_ROWS,)
    return pl.pallas_call(
        _pre_body,
        grid=grid,
        in_specs=[pl.BlockSpec((_PRE_ROWS, 128), lambda i: (i, 0))],
        out_specs=pl.BlockSpec((_PRE_ROWS, 128), lambda i: (i, 0)),
        out_shape=jax.ShapeDtypeStruct((NEL // 128, 128), jnp.int32),
    )(mask2d)


def _sc_body(flat_hbm, upd_hbm, zeros_hbm, out_hbm, idx_v0, val_v0, idx_v1,
             val_v1, zer_v, acc_sh, sem_in0, sem_in1, sem_out):
    s = lax.axis_index("c")
    t = lax.axis_index("s")
    idx_b = (idx_v0, idx_v1)
    val_b = (val_v0, val_v1)
    pltpu.sync_copy(zeros_hbm, zer_v)

    def _fire_chunk(q, j):
        c0 = SLOTS * (2 * q + s)
        h0 = NTEC * t + RUNS_PER_CHUNK * j
        sem = sem_in0 if j % 2 == 0 else sem_in1
        waits = []
        for k in range(RUNS_PER_CHUNK):
            src = (h0 + k) * (C * W) + c0 * W
            waits.append(pltpu.async_copy(
                flat_hbm.at[pl.ds(src, RUN)],
                idx_b[j % 2].at[pl.ds(k * RUN, RUN)], sem))
            waits.append(pltpu.async_copy(
                upd_hbm.at[pl.ds(src, RUN)],
                val_b[j % 2].at[pl.ds(k * RUN, RUN)], sem))
        return waits

    for q in range(PASSES):
        waits = [
            pltpu.async_copy(
                zer_v, acc_sh.at[pl.ds(t * ACC_SLICE + i * ZCHUNK, ZCHUNK)],
                sem_out)
            for i in range(ACC_SLICE // ZCHUNK)
        ]
        for wtt in waits:
            wtt.wait()
        plsc.subcore_barrier()
        waits = _fire_chunk(q, 0)
        for j in range(NCHUNK):
            for wtt in waits:
                wtt.wait()
            waits = _fire_chunk(q, j + 1) if j + 1 < NCHUNK else []
            pltpu.sync_copy(val_b[j % 2], acc_sh.at[idx_b[j % 2]], add=True)
        plsc.subcore_barrier()
        c0 = SLOTS * (2 * q + s)
        waits = []
        for k in range(DUMP_H):
            h2 = DUMP_H * t + k
            waits.append(pltpu.async_copy(
                acc_sh.at[pl.ds(h2 * DUMP_RUN, DUMP_RUN)],
                out_hbm.at[pl.ds(h2 * OROW + c0 * WO, DUMP_RUN)], sem_out))
        for wtt in waits:
            wtt.wait()


def _sc_scatter(flat, upd, zeros):
    mesh = plsc.VectorSubcoreMesh(core_axis_name="c", subcore_axis_name="s")
    fn = functools.partial(
        pl.kernel,
        mesh=mesh,
        out_type=jax.ShapeDtypeStruct((HO * C * WO,), jnp.float32),
        scratch_types=[
            pltpu.VMEM((CHUNK,), jnp.int32),
            pltpu.VMEM((CHUNK,), jnp.float32),
            pltpu.VMEM((CHUNK,), jnp.int32),
            pltpu.VMEM((CHUNK,), jnp.float32),
            pltpu.VMEM((ZCHUNK,), jnp.float32),
            pltpu.VMEM_SHARED((ACC_WORDS,), jnp.float32),
            pltpu.SemaphoreType.DMA,
            pltpu.SemaphoreType.DMA,
            pltpu.SemaphoreType.DMA,
        ],
    )(_sc_body)
    return fn(flat, upd, zeros)


def kernel(updates, mask):
    upd_n = updates.reshape(H, W, C).transpose(0, 2, 1).reshape(NEL)
    mask_n = (
        mask.astype(jnp.int32).reshape(H, W, C).transpose(0, 2, 1)
        .reshape(NEL // 128, 128)
    )
    flat = _pre(mask_n)
    zeros = jnp.zeros((ZCHUNK,), jnp.float32)
    out = _sc_scatter(flat.reshape(NEL), upd_n, zeros)
    return (
        out.reshape(HO, C, WO).transpose(0, 2, 1).reshape(1, HO, WO, C)
    )


# speedup vs baseline: 60.0801x; 1.2054x over previous
import functools

import jax
import jax.numpy as jnp
from jax import lax
from jax.experimental import pallas as pl
from jax.experimental.pallas import tpu as pltpu
from jax.experimental.pallas import tpu_sc as plsc

H = 256
W = 256
C = 96
NIN = H * W
NOUT = 4 * NIN
NEL = NIN * C
SLOTS = 6
PASSES = 8
NTEC = 16
RUN = SLOTS * W
RUNS_PER_CHUNK = 4
CHUNK = RUN * RUNS_PER_CHUNK
NCHUNK = 4
ACC_WORDS = SLOTS * NOUT
ACC_SLICE = ACC_WORDS // NTEC
ZCHUNK = 4096
WO = 2 * W
HO = 2 * H
OROW = C * WO
DUMP_RUN = SLOTS * WO
DUMP_H = HO // NTEC


def _pre_body(mask_ref, flat_ref):
    i = pl.program_id(0)
    m = mask_ref[...]
    r = lax.broadcasted_iota(jnp.int32, m.shape, 0) + i * _PRE_ROWS
    slot = (r // 2) % SLOTS
    p = ((m >> 5).astype(jnp.float32) * jnp.float32(0.33333334)).astype(jnp.int32)
    flat_ref[...] = (
        (p >> 9) * (SLOTS * WO)
        + ((p >> 7) & 3) * (SLOTS * 128)
        + slot * 128
        + (p & 127)
    )


_PRE_ROWS = 1536


def _pre(mask2d):
    grid = (NEL // 128 // _PRE_ROWS,)
    return pl.pallas_call(
        _pre_body,
        grid=grid,
        in_specs=[pl.BlockSpec((_PRE_ROWS, 128), lambda i: (i, 0))],
        out_specs=pl.BlockSpec((_PRE_ROWS, 128), lambda i: (i, 0)),
        out_shape=jax.ShapeDtypeStruct((NEL // 128, 128), jnp.int32),
    )(mask2d)


def _sc_body(flat_hbm, upd_hbm, zeros_hbm, out_hbm, idx_v0, val_v0, idx_v1,
             val_v1, zer_v, acc_sh, sem_in0, sem_in1, sem_out):
    s = lax.axis_index("c")
    t = lax.axis_index("s")
    idx_b = (idx_v0, idx_v1)
    val_b = (val_v0, val_v1)
    pltpu.sync_copy(zeros_hbm, zer_v)

    def _fire_chunk(q, j):
        c0 = SLOTS * (2 * q + s)
        h0 = NTEC * t + RUNS_PER_CHUNK * j
        sem = sem_in0 if j % 2 == 0 else sem_in1
        waits = []
        for k in range(RUNS_PER_CHUNK):
            src = (h0 + k) * (C * W) + c0 * W
            waits.append(pltpu.async_copy(
                flat_hbm.at[pl.ds(src, RUN)],
                idx_b[j % 2].at[pl.ds(k * RUN, RUN)], sem))
            waits.append(pltpu.async_copy(
                upd_hbm.at[pl.ds(src, RUN)],
                val_b[j % 2].at[pl.ds(k * RUN, RUN)], sem))
        return waits

    for q in range(PASSES):
        waits = [
            pltpu.async_copy(
                zer_v, acc_sh.at[pl.ds(t * ACC_SLICE + i * ZCHUNK, ZCHUNK)],
                sem_out)
            for i in range(ACC_SLICE // ZCHUNK)
        ]
        for wtt in waits:
            wtt.wait()
        plsc.subcore_barrier()
        waits = _fire_chunk(q, 0)
        for j in range(NCHUNK):
            for wtt in waits:
                wtt.wait()
            waits = _fire_chunk(q, j + 1) if j + 1 < NCHUNK else []
            pltpu.sync_copy(val_b[j % 2], acc_sh.at[idx_b[j % 2]], add=True)
        plsc.subcore_barrier()
        for sv in range(2):
            @pl.when(s == sv)
            def _dump():
                c0 = SLOTS * (2 * q + sv)
                ct, rem = c0 // 8, c0 % 8
                run1 = min(8 - rem, SLOTS) * 128

                def _row(k, _):
                    h2 = DUMP_H * t + k
                    a0 = h2 * DUMP_RUN
                    o0 = h2 * (C // 8 * 4096) + ct * 4096
                    for wt in range(4):
                        pltpu.async_copy(
                            acc_sh.at[pl.ds(a0 + wt * 768, run1)],
                            out_hbm.at[pl.ds(o0 + wt * 1024 + rem * 128, run1)],
                            sem_out)
                        if run1 < 768:
                            pltpu.async_copy(
                                acc_sh.at[pl.ds(a0 + wt * 768 + run1, 768 - run1)],
                                out_hbm.at[pl.ds(o0 + 4096 + wt * 1024, 768 - run1)],
                                sem_out)
                    return _

                lax.fori_loop(0, DUMP_H, _row, None)

        pltpu.make_async_copy(
            flat_hbm.at[pl.ds(0, ACC_SLICE)],
            acc_sh.at[pl.ds(t * ACC_SLICE, ACC_SLICE)],
            sem_out,
        ).wait()


def _sc_scatter(flat, upd, zeros):
    mesh = plsc.VectorSubcoreMesh(core_axis_name="c", subcore_axis_name="s")
    fn = functools.partial(
        pl.kernel,
        mesh=mesh,
        out_type=jax.ShapeDtypeStruct((HO * C * WO,), jnp.float32),
        scratch_types=[
            pltpu.VMEM((CHUNK,), jnp.int32),
            pltpu.VMEM((CHUNK,), jnp.float32),
            pltpu.VMEM((CHUNK,), jnp.int32),
            pltpu.VMEM((CHUNK,), jnp.float32),
            pltpu.VMEM((ZCHUNK,), jnp.float32),
            pltpu.VMEM_SHARED((ACC_WORDS,), jnp.float32),
            pltpu.SemaphoreType.DMA,
            pltpu.SemaphoreType.DMA,
            pltpu.SemaphoreType.DMA,
        ],
    )(_sc_body)
    return fn(flat, upd, zeros)


def kernel(updates, mask):
    upd_n = updates.reshape(H, W, C).transpose(0, 2, 1).reshape(NEL)
    mask_n = (
        mask.astype(jnp.int32).reshape(H, W, C).transpose(0, 2, 1)
        .reshape(NEL // 128, 128)
    )
    flat = _pre(mask_n)
    zeros = jnp.zeros((ZCHUNK,), jnp.float32)
    out = _sc_scatter(flat.reshape(NEL), upd_n, zeros)
    return (
        out.reshape(HO, C // 8, WO // 128, 8, 128)
        .transpose(0, 2, 4, 1, 3)
        .reshape(1, HO, WO, C)
    )


# speedup vs baseline: 61.9740x; 1.0315x over previous
import functools

import jax
import jax.numpy as jnp
from jax import lax
from jax.experimental import pallas as pl
from jax.experimental.pallas import tpu as pltpu
from jax.experimental.pallas import tpu_sc as plsc

H = 256
W = 256
C = 96
NIN = H * W
NOUT = 4 * NIN
NEL = NIN * C
SLOTS = 6
PASSES = 8
NTEC = 16
RUN = SLOTS * W
RUNS_PER_CHUNK = 4
CHUNK = RUN * RUNS_PER_CHUNK
NCHUNK = 4
ACC_WORDS = SLOTS * NOUT
ACC_SLICE = ACC_WORDS // NTEC
ZCHUNK = 4096
WO = 2 * W
HO = 2 * H
OROW = C * WO
DUMP_RUN = SLOTS * WO
DUMP_H = HO // NTEC


def _pre_body(mask_ref, flat_ref):
    i = pl.program_id(0)
    m = mask_ref[...]
    r = lax.broadcasted_iota(jnp.int32, m.shape, 0) + i * _PRE_ROWS
    slot = (r // 2) % SLOTS
    p = ((m >> 5).astype(jnp.float32) * jnp.float32(0.33333334)).astype(jnp.int32)
    flat_ref[...] = (
        (p >> 9) * (SLOTS * WO)
        + ((p >> 7) & 3) * (SLOTS * 128)
        + slot * 128
        + (p & 127)
    )


_PRE_ROWS = 1536


def _pre(mask2d):
    grid = (NEL // 128 // _PRE_ROWS,)
    return pl.pallas_call(
        _pre_body,
        grid=grid,
        in_specs=[pl.BlockSpec((_PRE_ROWS, 128), lambda i: (i, 0))],
        out_specs=pl.BlockSpec((_PRE_ROWS, 128), lambda i: (i, 0)),
        out_shape=jax.ShapeDtypeStruct((NEL // 128, 128), jnp.int32),
    )(mask2d)


def _sc_body(flat_hbm, upd_hbm, zeros_hbm, out_hbm, idx_v0, val_v0, idx_v1,
             val_v1, zer_v, acc_sh, sem_in0, sem_in1, sem_out):
    s = lax.axis_index("c")
    t = lax.axis_index("s")
    idx_b = (idx_v0, idx_v1)
    val_b = (val_v0, val_v1)
    pltpu.sync_copy(zeros_hbm, zer_v)

    def _fire_chunk(q, j):
        c0 = SLOTS * (2 * q + s)
        h0 = NTEC * t + RUNS_PER_CHUNK * j
        sem = sem_in0 if j % 2 == 0 else sem_in1
        waits = []
        for k in range(RUNS_PER_CHUNK):
            src = (h0 + k) * (C * W) + c0 * W
            waits.append(pltpu.async_copy(
                flat_hbm.at[pl.ds(src, RUN)],
                idx_b[j % 2].at[pl.ds(k * RUN, RUN)], sem))
            waits.append(pltpu.async_copy(
                upd_hbm.at[pl.ds(src, RUN)],
                val_b[j % 2].at[pl.ds(k * RUN, RUN)], sem))
        return waits

    for q in range(PASSES):
        in_waits = _fire_chunk(q, 0)
        waits = [
            pltpu.async_copy(
                zer_v, acc_sh.at[pl.ds(t * ACC_SLICE + i * ZCHUNK, ZCHUNK)],
                sem_out)
            for i in range(ACC_SLICE // ZCHUNK)
        ]
        for wtt in waits:
            wtt.wait()
        plsc.subcore_barrier()
        waits = in_waits
        for j in range(NCHUNK):
            for wtt in waits:
                wtt.wait()
            waits = _fire_chunk(q, j + 1) if j + 1 < NCHUNK else []
            pltpu.sync_copy(val_b[j % 2], acc_sh.at[idx_b[j % 2]], add=True)
        plsc.subcore_barrier()
        for sv in range(2):
            @pl.when(s == sv)
            def _dump():
                c0 = SLOTS * (2 * q + sv)
                ct, rem = c0 // 8, c0 % 8
                run1 = min(8 - rem, SLOTS) * 128

                def _row(k, _):
                    h2 = DUMP_H * t + k
                    a0 = h2 * DUMP_RUN
                    o0 = h2 * (C // 8 * 4096) + ct * 4096
                    for wt in range(4):
                        pltpu.async_copy(
                            acc_sh.at[pl.ds(a0 + wt * 768, run1)],
                            out_hbm.at[pl.ds(o0 + wt * 1024 + rem * 128, run1)],
                            sem_out)
                        if run1 < 768:
                            pltpu.async_copy(
                                acc_sh.at[pl.ds(a0 + wt * 768 + run1, 768 - run1)],
                                out_hbm.at[pl.ds(o0 + 4096 + wt * 1024, 768 - run1)],
                                sem_out)
                    return _

                lax.fori_loop(0, DUMP_H, _row, None)

        pltpu.make_async_copy(
            flat_hbm.at[pl.ds(0, ACC_SLICE)],
            acc_sh.at[pl.ds(t * ACC_SLICE, ACC_SLICE)],
            sem_out,
        ).wait()


def _sc_scatter(flat, upd, zeros):
    mesh = plsc.VectorSubcoreMesh(core_axis_name="c", subcore_axis_name="s")
    fn = functools.partial(
        pl.kernel,
        mesh=mesh,
        out_type=jax.ShapeDtypeStruct((HO * C * WO,), jnp.float32),
        scratch_types=[
            pltpu.VMEM((CHUNK,), jnp.int32),
            pltpu.VMEM((CHUNK,), jnp.float32),
            pltpu.VMEM((CHUNK,), jnp.int32),
            pltpu.VMEM((CHUNK,), jnp.float32),
            pltpu.VMEM((ZCHUNK,), jnp.float32),
            pltpu.VMEM_SHARED((ACC_WORDS,), jnp.float32),
            pltpu.SemaphoreType.DMA,
            pltpu.SemaphoreType.DMA,
            pltpu.SemaphoreType.DMA,
        ],
    )(_sc_body)
    return fn(flat, upd, zeros)


def kernel(updates, mask):
    upd_n = updates.reshape(H, W, C).transpose(0, 2, 1).reshape(NEL)
    mask_n = (
        mask.astype(jnp.int32).reshape(H, W, C).transpose(0, 2, 1)
        .reshape(NEL // 128, 128)
    )
    flat = _pre(mask_n)
    zeros = jnp.zeros((ZCHUNK,), jnp.float32)
    out = _sc_scatter(flat.reshape(NEL), upd_n, zeros)
    return (
        out.reshape(HO, C // 8, WO // 128, 8, 128)
        .transpose(0, 2, 4, 1, 3)
        .reshape(1, HO, WO, C)
    )
